# Initial kernel scaffold; baseline (speedup 1.0000x reference)
import jax
import jax.numpy as jnp
from jax.experimental import pallas as pl


def kernel(indexEmb, userFeatures, itemFeatures, user_table, item_table, W1, b1, W2, b2):
    raise NotImplementedError("write your pallas kernel here")


# speedup vs baseline: 1.6077x; 1.6077x over previous
import functools

import jax
import jax.numpy as jnp
from jax import lax
from jax.experimental import pallas as pl
from jax.experimental.pallas import tpu as pltpu
from jax.experimental.pallas import tpu_sc as plsc

B = 16384
VOCAB = 100000
EMB = 32
IDX_DIM = 64
HID = 256
U_FIELDS = 3
I_FIELDS = 4

NC = 2
NS = 16
NW = NC * NS

NU = B * U_FIELDS
NI = B * I_FIELDS
BU = NU // NW
BI = NI // NW
CHUNK = 128


def _sc_gather_body(u_tab, i_tab, u_idx, i_idx, u_out, i_out,
                    idx_v, rows_v, sem):
    wid = lax.axis_index("s") * NC + lax.axis_index("c")

    def phase(tab_hbm, idx_hbm, out_hbm, n):
        base = wid * n
        pltpu.sync_copy(idx_hbm.at[pl.ds(base, n)], idx_v.at[pl.ds(0, n)])
        copies = []
        for c in range(n // CHUNK):
            copies.append(pltpu.async_copy(
                tab_hbm.at[idx_v.at[pl.ds(c * CHUNK, CHUNK)]],
                rows_v.at[pl.ds(c * CHUNK, CHUNK)],
                sem,
            ))
        for cp in copies:
            cp.wait()
        pltpu.sync_copy(rows_v.at[pl.ds(0, n)], out_hbm.at[pl.ds(base, n)])

    phase(u_tab, u_idx, u_out, BU)
    phase(i_tab, i_idx, i_out, BI)


def _mlp_body(idx_ref, u_ref, it_ref, w1_ref, b1_ref, w2_ref, b2_ref, o_ref):
    hp = jax.lax.Precision.HIGHEST
    h = jnp.dot(idx_ref[...], w1_ref[0:IDX_DIM, :],
                preferred_element_type=jnp.float32, precision=hp)
    h += jnp.dot(u_ref[...], w1_ref[IDX_DIM:IDX_DIM + U_FIELDS * EMB, :],
                 preferred_element_type=jnp.float32, precision=hp)
    h += jnp.dot(it_ref[...], w1_ref[IDX_DIM + U_FIELDS * EMB:, :],
                 preferred_element_type=jnp.float32, precision=hp)
    h = jnp.maximum(h + b1_ref[...], 0.0)
    logits = jnp.dot(h, w2_ref[...],
                     preferred_element_type=jnp.float32, precision=hp)
    logits += b2_ref[...]
    m = jnp.max(logits, axis=-1, keepdims=True)
    e = jnp.exp(logits - m)
    o_ref[...] = e / jnp.sum(e, axis=-1, keepdims=True)


_MLP_BLK = 2048


def kernel(indexEmb, userFeatures, itemFeatures, user_table, item_table,
           W1, b1, W2, b2):
    u_tab = user_table.reshape(U_FIELDS * VOCAB, EMB)
    i_tab = item_table.reshape(I_FIELDS * VOCAB, EMB)
    u_off = jnp.arange(U_FIELDS, dtype=jnp.int32) * VOCAB
    i_off = jnp.arange(I_FIELDS, dtype=jnp.int32) * VOCAB
    u_idx = (userFeatures + u_off[None, :]).reshape(NU)
    i_idx = (itemFeatures + i_off[None, :]).reshape(NI)

    mesh = plsc.VectorSubcoreMesh(core_axis_name="c", subcore_axis_name="s")
    sc_gather = pl.kernel(
        _sc_gather_body,
        out_type=(
            jax.ShapeDtypeStruct((NU, EMB), jnp.float32),
            jax.ShapeDtypeStruct((NI, EMB), jnp.float32),
        ),
        mesh=mesh,
        scratch_types=[
            pltpu.VMEM((BI,), jnp.int32),
            pltpu.VMEM((BI, EMB), jnp.float32),
            pltpu.SemaphoreType.DMA,
        ],
        compiler_params=pltpu.CompilerParams(use_tc_tiling_on_sc=False),
    )
    u_rows, i_rows = sc_gather(u_tab, i_tab, u_idx, i_idx)

    u_feat = u_rows.reshape(B, U_FIELDS * EMB)
    i_feat = i_rows.reshape(B, I_FIELDS * EMB)

    grid = (B // _MLP_BLK,)
    out = pl.pallas_call(
        _mlp_body,
        grid=grid,
        in_specs=[
            pl.BlockSpec((_MLP_BLK, IDX_DIM), lambda i: (i, 0)),
            pl.BlockSpec((_MLP_BLK, U_FIELDS * EMB), lambda i: (i, 0)),
            pl.BlockSpec((_MLP_BLK, I_FIELDS * EMB), lambda i: (i, 0)),
            pl.BlockSpec((IDX_DIM + 7 * EMB, HID), lambda i: (0, 0)),
            pl.BlockSpec((1, HID), lambda i: (0, 0)),
            pl.BlockSpec((HID, 2), lambda i: (0, 0)),
            pl.BlockSpec((1, 2), lambda i: (0, 0)),
        ],
        out_specs=pl.BlockSpec((_MLP_BLK, 2), lambda i: (i, 0)),
        out_shape=jax.ShapeDtypeStruct((B, 2), jnp.float32),
    )(indexEmb, u_feat, i_feat, W1, b1.reshape(1, HID), W2, b2.reshape(1, 2))
    return out
